# baseline (speedup 1.0000x reference)
import functools

import numpy as np
import jax
import jax.numpy as jnp
from jax import lax
from jax.experimental import pallas as pl
from jax.experimental.pallas import tpu as pltpu
from jax.experimental.pallas import tpu_sc as plsc

_MASK_INDEXES = (1, 4, 7, 10, 13)
_PPI = 64

_consts_cache = {}


def _rotl32(x, r):
    return ((x << np.uint32(r)) | (x >> np.uint32(32 - r))).astype(np.uint32)


def _threefry2x32(k0, k1, x0, x1):
    ks0 = np.uint32(k0)
    ks1 = np.uint32(k1)
    ks2 = np.uint32(ks0 ^ ks1 ^ np.uint32(0x1BD11BDA))
    x0 = (x0 + ks0).astype(np.uint32)
    x1 = (x1 + ks1).astype(np.uint32)
    rots = ((13, 15, 26, 6), (17, 29, 16, 24))
    ks = (ks0, ks1, ks2)
    for i in range(5):
        for r in rots[i % 2]:
            x0 = (x0 + x1).astype(np.uint32)
            x1 = _rotl32(x1, r)
            x1 = (x1 ^ x0).astype(np.uint32)
        x0 = (x0 + ks[(i + 1) % 3]).astype(np.uint32)
        x1 = (x1 + ks[(i + 2) % 3] + np.uint32(i + 1)).astype(np.uint32)
    return x0, x1


def _uniform_key42(shape):
    n = int(np.prod(shape))
    i64 = np.arange(n, dtype=np.uint64)
    c_hi = (i64 >> np.uint64(32)).astype(np.uint32)
    c_lo = (i64 & np.uint64(0xFFFFFFFF)).astype(np.uint32)
    b0, b1 = _threefry2x32(0, 42, c_hi, c_lo)
    bits = (b0 ^ b1).astype(np.uint32)
    f = ((bits >> np.uint32(9)) | np.uint32(0x3F800000)).view(np.float32)
    f = f - np.float32(1.0)
    return np.maximum(np.float32(0.0), f).reshape(shape)


def _constants(N, L):
    ck = (N, L)
    if ck not in _consts_cache:
        noise = _uniform_key42((N, L))
        masked_pos = np.array(
            [idx * _PPI + i for idx in _MASK_INDEXES for i in range(_PPI)],
            dtype=np.int64,
        )
        noise[:, masked_pos] = 2.0
        len_keep = L - masked_pos.size
        ids_shuffle = np.argsort(noise, axis=1, kind="stable").astype(np.int32)
        ids_restore = np.argsort(ids_shuffle, axis=1, kind="stable").astype(np.int32)
        ids_keep = ids_shuffle[:, :len_keep]
        mask = np.ones((N, L), dtype=np.float32)
        mask[:, :len_keep] = 0.0
        mask = np.take_along_axis(mask, ids_restore, axis=1)
        gidx = (ids_keep.astype(np.int64) + np.arange(N, dtype=np.int64)[:, None] * L)
        gidx = gidx.astype(np.int32).reshape(-1)
        _consts_cache[ck] = (gidx, mask, ids_restore, len_keep)
    return _consts_cache[ck]


def _make_gather(num_rows_total, B, D, NC, NS):
    NW = NC * NS
    b_per_w = B // NW
    R = 128
    C = b_per_w // R
    DP = 256
    mesh = plsc.VectorSubcoreMesh(core_axis_name="c", subcore_axis_name="s")

    @functools.partial(
        pl.kernel,
        mesh=mesh,
        compiler_params=pltpu.CompilerParams(use_tc_tiling_on_sc=True),
        out_type=jax.ShapeDtypeStruct((B, DP), jnp.float32),
        scratch_types=[
            pltpu.VMEM((C, R), jnp.int32),
            pltpu.VMEM((R, DP), jnp.float32),
            pltpu.VMEM((R, DP), jnp.float32),
            pltpu.SemaphoreType.DMA,
            pltpu.SemaphoreType.DMA,
            pltpu.SemaphoreType.DMA,
            pltpu.SemaphoreType.DMA,
        ],
    )
    def gather_k(table_hbm, idx_hbm, out_hbm, idx_v, buf0, buf1,
                 gsem0, gsem1, ssem0, ssem1):
        wid = lax.axis_index("s") * NC + lax.axis_index("c")
        base = wid * b_per_w
        bufs = (buf0, buf1)
        gsems = (gsem0, gsem1)
        ssems = (ssem0, ssem1)
        pltpu.sync_copy(idx_hbm.at[wid], idx_v)
        gathers = {}
        stores = {}
        gathers[0] = pltpu.async_copy(
            table_hbm.at[idx_v.at[0]], bufs[0], gsems[0])
        for c in range(C):
            b = c % 2
            nxt = c + 1
            if nxt < C:
                nb = nxt % 2
                if nxt >= 2:
                    stores[nxt - 2].wait()
                gathers[nxt] = pltpu.async_copy(
                    table_hbm.at[idx_v.at[nxt]], bufs[nb], gsems[nb])
            gathers[c].wait()
            stores[c] = pltpu.async_copy(
                bufs[b], out_hbm.at[pl.ds(base + c * R, R)], ssems[b])
        stores[C - 2].wait()
        stores[C - 1].wait()

    return gather_k


def kernel(x):
    N, L, D = x.shape
    gidx, mask, ids_restore, len_keep = _constants(N, L)
    info = plsc.get_sparse_core_info()
    NC, NS = info.num_cores, info.num_subcores
    NH = N // 2
    gidx2 = gidx.reshape(N, len_keep)
    outs = []
    for h in range(2):
        xh = lax.slice(x, (h * NH, 0, 0), ((h + 1) * NH, L, D))
        xph = jnp.pad(xh, ((0, 0), (0, 0), (0, 256 - D))).reshape(NH * L, 256)
        gh = gidx2[h * NH:(h + 1) * NH] - np.int32(h * NH * L)
        idx3h = jnp.asarray(gh.reshape(NC * NS, -1, 128))
        outs.append(
            _make_gather(NH * L, NH * len_keep, D, NC, NS)(xph, idx3h))
    out = jnp.concatenate(outs, axis=0)
    B = N * len_keep
    x_masked = lax.slice(out, (0, 0), (B, D)).reshape(N, len_keep, D)
    return (x_masked, jnp.asarray(mask), jnp.asarray(ids_restore))


# speedup vs baseline: 1.4040x; 1.4040x over previous
import functools

import numpy as np
import jax
import jax.numpy as jnp
from jax import lax
from jax.experimental import pallas as pl
from jax.experimental.pallas import tpu as pltpu
from jax.experimental.pallas import tpu_sc as plsc

_MASK_INDEXES = (1, 4, 7, 10, 13)
_PPI = 64

_consts_cache = {}


def _rotl32(x, r):
    return ((x << np.uint32(r)) | (x >> np.uint32(32 - r))).astype(np.uint32)


def _threefry2x32(k0, k1, x0, x1):
    ks0 = np.uint32(k0)
    ks1 = np.uint32(k1)
    ks2 = np.uint32(ks0 ^ ks1 ^ np.uint32(0x1BD11BDA))
    x0 = (x0 + ks0).astype(np.uint32)
    x1 = (x1 + ks1).astype(np.uint32)
    rots = ((13, 15, 26, 6), (17, 29, 16, 24))
    ks = (ks0, ks1, ks2)
    for i in range(5):
        for r in rots[i % 2]:
            x0 = (x0 + x1).astype(np.uint32)
            x1 = _rotl32(x1, r)
            x1 = (x1 ^ x0).astype(np.uint32)
        x0 = (x0 + ks[(i + 1) % 3]).astype(np.uint32)
        x1 = (x1 + ks[(i + 2) % 3] + np.uint32(i + 1)).astype(np.uint32)
    return x0, x1


def _uniform_key42(shape):
    n = int(np.prod(shape))
    i64 = np.arange(n, dtype=np.uint64)
    c_hi = (i64 >> np.uint64(32)).astype(np.uint32)
    c_lo = (i64 & np.uint64(0xFFFFFFFF)).astype(np.uint32)
    b0, b1 = _threefry2x32(0, 42, c_hi, c_lo)
    bits = (b0 ^ b1).astype(np.uint32)
    f = ((bits >> np.uint32(9)) | np.uint32(0x3F800000)).view(np.float32)
    f = f - np.float32(1.0)
    return np.maximum(np.float32(0.0), f).reshape(shape)


def _constants(N, L):
    ck = (N, L)
    if ck not in _consts_cache:
        noise = _uniform_key42((N, L))
        masked_pos = np.array(
            [idx * _PPI + i for idx in _MASK_INDEXES for i in range(_PPI)],
            dtype=np.int64,
        )
        noise[:, masked_pos] = 2.0
        len_keep = L - masked_pos.size
        ids_shuffle = np.argsort(noise, axis=1, kind="stable").astype(np.int32)
        ids_restore = np.argsort(ids_shuffle, axis=1, kind="stable").astype(np.int32)
        ids_keep = ids_shuffle[:, :len_keep]
        mask = np.ones((N, L), dtype=np.float32)
        mask[:, :len_keep] = 0.0
        mask = np.take_along_axis(mask, ids_restore, axis=1)
        gidx = (ids_keep.astype(np.int64) + np.arange(N, dtype=np.int64)[:, None] * L)
        gidx = gidx.astype(np.int32).reshape(-1)
        _consts_cache[ck] = (gidx, mask, ids_restore, len_keep)
    return _consts_cache[ck]


def _make_gather(num_rows_total, B, D, NC, NS):
    NW = NC * NS
    b_per_w = B // NW
    R = 128
    C = b_per_w // R
    DP = 256
    mesh = plsc.VectorSubcoreMesh(core_axis_name="c", subcore_axis_name="s")

    @functools.partial(
        pl.kernel,
        mesh=mesh,
        compiler_params=pltpu.CompilerParams(use_tc_tiling_on_sc=True),
        out_type=jax.ShapeDtypeStruct((B, DP), jnp.float32),
        scratch_types=[
            pltpu.VMEM((C, R), jnp.int32),
            pltpu.VMEM((R, DP), jnp.float32),
            pltpu.VMEM((R, DP), jnp.float32),
            pltpu.VMEM((R, DP), jnp.float32),
            pltpu.SemaphoreType.DMA,
            pltpu.SemaphoreType.DMA,
            pltpu.SemaphoreType.DMA,
            pltpu.SemaphoreType.DMA,
            pltpu.SemaphoreType.DMA,
            pltpu.SemaphoreType.DMA,
        ],
    )
    def gather_k(table_hbm, idx_hbm, out_hbm, idx_v, buf0, buf1, buf2,
                 gsem0, gsem1, gsem2, ssem0, ssem1, ssem2):
        wid = lax.axis_index("s") * NC + lax.axis_index("c")
        base = wid * b_per_w
        bufs = (buf0, buf1, buf2)
        gsems = (gsem0, gsem1, gsem2)
        ssems = (ssem0, ssem1, ssem2)
        NB = 3
        pltpu.sync_copy(idx_hbm.at[wid], idx_v)
        gathers = {}
        stores = {}
        for p in range(min(NB - 1, C)):
            gathers[p] = pltpu.async_copy(
                table_hbm.at[idx_v.at[p]], bufs[p], gsems[p])
        for c in range(C):
            b = c % NB
            nxt = c + NB - 1
            if nxt < C:
                nb = nxt % NB
                if nxt >= NB:
                    stores[nxt - NB].wait()
                gathers[nxt] = pltpu.async_copy(
                    table_hbm.at[idx_v.at[nxt]], bufs[nb], gsems[nb])
            gathers[c].wait()
            stores[c] = pltpu.async_copy(
                bufs[b], out_hbm.at[pl.ds(base + c * R, R)], ssems[b])
        for c in range(max(0, C - NB), C):
            stores[c].wait()

    return gather_k


def kernel(x):
    N, L, D = x.shape
    gidx, mask, ids_restore, len_keep = _constants(N, L)
    B = N * len_keep
    info = plsc.get_sparse_core_info()
    NC, NS = info.num_cores, info.num_subcores
    xp = jnp.pad(x, ((0, 0), (0, 0), (0, 256 - D))).reshape(N * L, 256)
    idx3 = jnp.asarray(gidx.reshape(NC * NS, -1, 128))
    out = _make_gather(N * L, B, D, NC, NS)(xp, idx3)
    x_masked = lax.slice(out, (0, 0), (B, D)).reshape(N, len_keep, D)
    return (x_masked, jnp.asarray(mask), jnp.asarray(ids_restore))
